# baseline (device time: 18319 ns/iter reference)
import jax
import jax.numpy as jnp
from jax import lax
from jax.experimental import pallas as pl
from jax.experimental.pallas import tpu as pltpu

N_DEV = 4
N_LAYERS = 3
BF = jnp.bfloat16


def kernel(x, Win0, Wout0, Win1, Wout1, Win2, Wout2):
    b, d = x.shape
    out_rows = b // N_DEV

    def mm(a, w):
        return jnp.dot(a, w, preferred_element_type=jnp.float32)

    def body(x_ref, win0, wout0, win1, wout1, win2, wout2,
             out_ref, send_buf, comm_ref, rs_ref, own_ref, send_sems, recv_sems):
        my = lax.axis_index("i")

        wins = [win0[:, :].astype(BF), win1[:, :].astype(BF), win2[:, :].astype(BF)]
        wouts = [wout0[:, :].astype(BF), wout1[:, :].astype(BF), wout2[:, :].astype(BF)]

        h = jnp.maximum(mm(x_ref[:, :].astype(BF), wins[0]), 0.0)
        partial = mm(h.astype(BF), wouts[0])
        send_buf[0] = partial.astype(BF)

        barrier = pltpu.get_barrier_semaphore()
        for nbr in [(my + 1) % N_DEV, (my - 1) % N_DEV]:
            pl.semaphore_signal(
                barrier, inc=1,
                device_id=(nbr,), device_id_type=pl.DeviceIdType.MESH,
            )
        pl.semaphore_wait(barrier, 2)

        pending = []
        for l in range(N_LAYERS):
            last = l == N_LAYERS - 1
            for off in (2, 1, 3):
                peer = (my + off) % N_DEV
                if last:
                    src = send_buf.at[l, pl.ds(peer * out_rows, out_rows)]
                    dst = rs_ref.at[off - 1]
                else:
                    src = send_buf.at[l]
                    dst = comm_ref.at[l, off - 1]
                rdma = pltpu.make_async_remote_copy(
                    src_ref=src,
                    dst_ref=dst,
                    send_sem=send_sems.at[l, off - 1],
                    recv_sem=recv_sems.at[l, off - 1],
                    device_id=(peer,),
                    device_id_type=pl.DeviceIdType.MESH,
                )
                rdma.start()
                pending.append(rdma)

            if last:
                own_ref[:, :] = partial

            if last:
                acc = own_ref[pl.ds(my * out_rows, out_rows), :]
            else:
                acc = mm(send_buf[l], wins[l + 1])
            for off in (1, 3, 2):
                if last:
                    src = send_buf.at[l, pl.ds(my * out_rows, out_rows)]
                    dst = rs_ref.at[off - 1]
                else:
                    src = send_buf.at[l]
                    dst = comm_ref.at[l, off - 1]
                recv = pltpu.make_async_remote_copy(
                    src_ref=src,
                    dst_ref=dst,
                    send_sem=send_sems.at[l, off - 1],
                    recv_sem=recv_sems.at[l, off - 1],
                    device_id=((my + off) % N_DEV,),
                    device_id_type=pl.DeviceIdType.MESH,
                )
                recv.wait_recv()
                if last:
                    acc = acc + rs_ref[off - 1].astype(jnp.float32)
                else:
                    acc = acc + mm(comm_ref[l, off - 1], wins[l + 1])
            if last:
                out_ref[:, :] = acc
            else:
                h = jnp.maximum(acc, 0.0)
                partial = mm(h.astype(BF), wouts[l + 1])
                send_buf[l + 1] = partial.astype(BF)

        for rdma in pending:
            rdma.wait_send()

    return pl.pallas_call(
        body,
        out_shape=jax.ShapeDtypeStruct((out_rows, d), jnp.float32),
        in_specs=[pl.BlockSpec(memory_space=pltpu.VMEM)] * 7,
        out_specs=pl.BlockSpec(memory_space=pltpu.VMEM),
        scratch_shapes=[
            pltpu.VMEM((N_LAYERS, b, d), BF),
            pltpu.VMEM((N_LAYERS - 1, N_DEV - 1, b, d), BF),
            pltpu.VMEM((N_DEV - 1, out_rows, d), BF),
            pltpu.VMEM((b, d), jnp.float32),
            pltpu.SemaphoreType.DMA((N_LAYERS, N_DEV - 1)),
            pltpu.SemaphoreType.DMA((N_LAYERS, N_DEV - 1)),
        ],
        compiler_params=pltpu.CompilerParams(collective_id=0),
    )(x, Win0, Wout0, Win1, Wout1, Win2, Wout2)
